# Initial kernel scaffold; baseline (speedup 1.0000x reference)
import jax
import jax.numpy as jnp
from jax.experimental import pallas as pl


def kernel(past_values, past_time_features):
    raise NotImplementedError("write your pallas kernel here")


# speedup vs baseline: 87.1626x; 87.1626x over previous
import numpy as np
import jax
import jax.numpy as jnp
from jax.experimental import pallas as pl

_UNIQUE_WL = jnp.array([3670.69, 4826.85, 6223.24, 7545.98, 8590.9, 9710.28],
                       dtype=jnp.float32)
_K = 7
_N = 300
_NUM = 128 * 6
_MAX_ITER = 100

_PERM = np.asarray(
    jax.vmap(lambda k: jax.random.permutation(k, _N)[:_K])(
        jax.random.split(jax.random.key(42), _NUM)))


def _kmeans_body(x_ref, c0_ref, out_ref):
    x = x_ref[:, :]
    cent0 = c0_ref[:, :]

    def step(_, carry):
        cent, _lab = carry
        mind = jnp.full((_NUM, _N), jnp.inf, dtype=jnp.float32)
        lab = jnp.zeros((_NUM, _N), dtype=jnp.int32)
        for c in range(_K):
            d = jnp.abs(x - cent[:, c:c + 1])
            m = d < mind
            mind = jnp.where(m, d, mind)
            lab = jnp.where(m, c, lab)
        cols = []
        for c in range(_K):
            mc = lab == c
            s = jnp.sum(jnp.where(mc, x, 0.0), axis=1, keepdims=True)
            n = jnp.sum(jnp.where(mc, 1.0, 0.0), axis=1, keepdims=True)
            cols.append(jnp.where(n > 0, s / jnp.maximum(n, 1.0),
                                  cent[:, c:c + 1]))
        new_cent = jnp.concatenate(cols, axis=1)
        return new_cent, lab

    _, lab = jax.lax.fori_loop(
        0, _MAX_ITER, step,
        (cent0, jnp.zeros((_NUM, _N), dtype=jnp.int32)))

    for c in range(_K):
        out_ref[c, :, :] = (lab == c).astype(jnp.float32)


def kernel(past_values, past_time_features):
    wl = past_time_features[:, :, 1]
    err = past_values[:, :, 1]
    series = jnp.where(wl[:, None, :] == _UNIQUE_WL[None, :, None],
                       err[:, None, :], jnp.float32(0.0))
    x = series.reshape(_NUM, _N)
    cent0 = jnp.take_along_axis(x, jnp.asarray(_PERM), axis=1)

    out = pl.pallas_call(
        _kmeans_body,
        out_shape=jax.ShapeDtypeStruct((_K, _NUM, _N), jnp.float32),
    )(x, cent0)
    return jnp.transpose(out, (1, 0, 2)).astype(bool)


# speedup vs baseline: 285.8957x; 3.2800x over previous
import numpy as np
import jax
import jax.numpy as jnp
from jax.experimental import pallas as pl

_UNIQUE_WL = jnp.array([3670.69, 4826.85, 6223.24, 7545.98, 8590.9, 9710.28],
                       dtype=jnp.float32)
_K = 7
_N = 300
_NUM = 128 * 6
_MAX_ITER = 100

_PERM = np.asarray(
    jax.vmap(lambda k: jax.random.permutation(k, _N)[:_K])(
        jax.random.split(jax.random.key(42), _NUM)))


def _kmeans_body(x_ref, c0_ref, out_ref):
    x = x_ref[:, :]
    cent0 = c0_ref[:, :]

    def cond(carry):
        t, _cent, _lab, conv = carry
        return jnp.logical_and(t < _MAX_ITER, jnp.logical_not(conv))

    def step(carry):
        t, cent, _lab, _conv = carry
        mind = jnp.full((_NUM, _N), jnp.inf, dtype=jnp.float32)
        lab = jnp.zeros((_NUM, _N), dtype=jnp.int32)
        for c in range(_K):
            d = jnp.abs(x - cent[:, c:c + 1])
            m = d < mind
            mind = jnp.where(m, d, mind)
            lab = jnp.where(m, c, lab)
        cols = []
        for c in range(_K):
            mc = lab == c
            s = jnp.sum(jnp.where(mc, x, 0.0), axis=1, keepdims=True)
            n = jnp.sum(jnp.where(mc, 1.0, 0.0), axis=1, keepdims=True)
            cols.append(jnp.where(n > 0, s / jnp.maximum(n, 1.0),
                                  cent[:, c:c + 1]))
        new_cent = jnp.concatenate(cols, axis=1)
        conv = jnp.all(new_cent == cent)
        return t + 1, new_cent, lab, conv

    _, _, lab, _ = jax.lax.while_loop(
        cond, step,
        (jnp.int32(0), cent0, jnp.zeros((_NUM, _N), dtype=jnp.int32),
         jnp.bool_(False)))

    for c in range(_K):
        out_ref[c, :, :] = (lab == c).astype(jnp.float32)


def kernel(past_values, past_time_features):
    wl = past_time_features[:, :, 1]
    err = past_values[:, :, 1]
    series = jnp.where(wl[:, None, :] == _UNIQUE_WL[None, :, None],
                       err[:, None, :], jnp.float32(0.0))
    x = series.reshape(_NUM, _N)
    cent0 = jnp.take_along_axis(x, jnp.asarray(_PERM), axis=1)

    out = pl.pallas_call(
        _kmeans_body,
        out_shape=jax.ShapeDtypeStruct((_K, _NUM, _N), jnp.float32),
    )(x, cent0)
    return jnp.transpose(out, (1, 0, 2)).astype(bool)
